# baseline (speedup 1.0000x reference)
import functools

import jax
import jax.numpy as jnp
from jax import lax
from jax.experimental import pallas as pl
from jax.experimental.pallas import tpu as pltpu
from jax.experimental.pallas import tpu_sc as plsc

N = 500000
FEAT = 20
HID = 64
S = 16
B = 16384
U = 2000
NU = N // U
NW = 32


def _unit(v):
    mag = jnp.maximum(jnp.sqrt(jnp.sum(v * v, axis=1)), 1e-8)
    return v / mag[:, None]


def _slot_mats(rotation, translation):
    ident = jnp.array([[1.0, 0.0, 0.0, 0.0, 1.0, 0.0]], jnp.float32)
    rot6 = jnp.concatenate([ident, rotation[1:]], axis=0)
    tr = jnp.concatenate([jnp.zeros((1, 3), jnp.float32), translation[1:]], axis=0)
    x = _unit(rot6[:, 0:3])
    z = _unit(jnp.cross(x, rot6[:, 3:6]))
    y = jnp.cross(z, x)
    rot = jnp.stack([x, y, z], axis=-1)
    rmc = jnp.transpose(rot, (2, 0, 1)).reshape(3 * S, 3)
    trc = tr.T.reshape(3 * S, 1)
    rft = rot.reshape(S, 9).T
    trt = tr.T
    return rmc, trc, rft, trt


def _tc_body(xt_ref, et_ref, w1et_ref, w1xt_ref, b1t_ref, w2t_ref, b2t_ref,
             w3t_ref, b3t_ref, rmc_ref, trc_ref, rft_ref, trt_ref,
             soft_ref, pred_ref, cand_ref):
    xt = xt_ref[...]
    et = et_ref[...]
    h = jnp.dot(w1et_ref[...], et, preferred_element_type=jnp.float32)
    h = h + jnp.dot(w1xt_ref[...], xt, preferred_element_type=jnp.float32)
    h = jax.nn.relu(h + b1t_ref[...])
    h = jax.nn.relu(jnp.dot(w2t_ref[...], h, preferred_element_type=jnp.float32)
                    + b2t_ref[...])
    lt = jnp.dot(w3t_ref[...], h, preferred_element_type=jnp.float32) + b3t_ref[...]

    m = jnp.max(lt, axis=0, keepdims=True)
    ex = jnp.exp(lt - m)
    soft_ref[...] = ex * (1.0 / jnp.sum(ex, axis=0, keepdims=True))

    cand_ref[...] = (jnp.dot(rmc_ref[...], xt, preferred_element_type=jnp.float32)
                     + trc_ref[...])

    srow = jax.lax.broadcasted_iota(jnp.int32, (S, B), 0)
    ismax = lt >= m
    idx = jnp.min(jnp.where(ismax, srow, S), axis=0, keepdims=True)
    hard_t = (srow == idx).astype(jnp.float32)
    rotsel = jnp.dot(rft_ref[...], hard_t, preferred_element_type=jnp.float32)
    pred = jnp.dot(trt_ref[...], hard_t, preferred_element_type=jnp.float32)
    for d in range(3):
        pred = pred + (jnp.broadcast_to(xt[d:d + 1, :], (3, B))
                       * rotsel[3 * d:3 * d + 3, :])
    pred_ref[...] = pred


def _tc_call(xt, et, w1et, w1xt, b1t, w2t, b2t, w3t, b3t, rmc, trc, rft, trt):
    grid = (pl.cdiv(N, B),)

    def data_spec(rows):
        return pl.BlockSpec((rows, B), lambda i: (0, i))

    def full_spec(shape):
        return pl.BlockSpec(shape, lambda i: (0,) * len(shape))

    return pl.pallas_call(
        _tc_body,
        grid=grid,
        in_specs=[
            data_spec(3),
            data_spec(FEAT),
            full_spec((HID, FEAT)),
            full_spec((HID, 3)),
            full_spec((HID, 1)),
            full_spec((HID, HID)),
            full_spec((HID, 1)),
            full_spec((S, HID)),
            full_spec((S, 1)),
            full_spec((3 * S, 3)),
            full_spec((3 * S, 1)),
            full_spec((9, S)),
            full_spec((3, S)),
        ],
        out_specs=[data_spec(S), data_spec(3), data_spec(3 * S)],
        out_shape=[
            jax.ShapeDtypeStruct((S, N), jnp.float32),
            jax.ShapeDtypeStruct((3, N), jnp.float32),
            jax.ShapeDtypeStruct((3 * S, N), jnp.float32),
        ],
        compiler_params=pltpu.CompilerParams(
            dimension_semantics=("parallel",),
        ),
    )(xt, et, w1et, w1xt, b1t, w2t, b2t, w3t, b3t, rmc, trc, rft, trt)


def _sc_body(soft_hbm, hard_hbm, soft_v, hard_v):
    wid = lax.axis_index("s") * 2 + lax.axis_index("c")

    def do_unit(k, carry):
        u = wid + k * NW

        @pl.when(u < NU)
        def _process():
            off = u * U
            for s in range(S):
                pltpu.sync_copy(soft_hbm.at[pl.ds(s * N + off, U)],
                                soft_v.at[pl.ds(s * U, U)])

            def do_group(j, gcarry):
                o = j * 16
                m = soft_v[pl.ds(o, 16)]
                idx = jnp.zeros((16,), jnp.int32)
                for s in range(1, S):
                    v = soft_v[pl.ds(s * U + o, 16)]
                    gt = v > m
                    m = jnp.where(gt, v, m)
                    idx = jnp.where(gt, s, idx)
                for s in range(S):
                    hard_v[pl.ds(s * U + o, 16)] = jnp.where(
                        idx == s, 1.0, 0.0).astype(jnp.float32)
                return gcarry

            lax.fori_loop(0, U // 16, do_group, 0)

            for s in range(S):
                pltpu.sync_copy(hard_v.at[pl.ds(s * U, U)],
                                hard_hbm.at[pl.ds(s * N + off, U)])

        return carry

    lax.fori_loop(0, (NU + NW - 1) // NW, do_unit, 0)


def _sc_call(soft):
    mesh = plsc.VectorSubcoreMesh(core_axis_name="c", subcore_axis_name="s")
    f = pl.kernel(
        _sc_body,
        mesh=mesh,
        out_type=[
            jax.ShapeDtypeStruct((S * N,), jnp.float32),
        ],
        scratch_types=[
            pltpu.VMEM((S * U,), jnp.float32),
            pltpu.VMEM((S * U,), jnp.float32),
        ],
    )
    return f(soft.reshape(S * N))


def kernel(xyz_cnc, xyz_cnc_embedded, W1, b1, W2, b2, W3, b3, rotation, translation):
    rmc, trc, rft, trt = _slot_mats(rotation, translation)
    xt = xyz_cnc.T
    soft_t, pred_t, cand_c = _tc_call(
        xt, xyz_cnc_embedded.T, W1[:FEAT].T, W1[FEAT:].T,
        b1.reshape(HID, 1), W2.T, b2.reshape(HID, 1), W3.T, b3.reshape(S, 1),
        rmc, trc, rft, trt)
    (hard_f,) = _sc_call(soft_t)

    attn_hard = hard_f.reshape(S, N).T
    attn_soft = soft_t.T
    pred = pred_t.T
    cand = jnp.transpose(cand_c.reshape(3, S, N), (2, 1, 0))
    return (attn_hard, attn_soft, pred, cand)


# speedup vs baseline: 11.3796x; 11.3796x over previous
import jax
import jax.numpy as jnp
from jax.experimental import pallas as pl
from jax.experimental.pallas import tpu as pltpu

N = 500000
FEAT = 20
HID = 64
S = 16
B = 16384


def _unit(v):
    mag = jnp.maximum(jnp.sqrt(jnp.sum(v * v, axis=1)), 1e-8)
    return v / mag[:, None]


def _slot_mats(rotation, translation):
    ident = jnp.array([[1.0, 0.0, 0.0, 0.0, 1.0, 0.0]], jnp.float32)
    rot6 = jnp.concatenate([ident, rotation[1:]], axis=0)
    tr = jnp.concatenate([jnp.zeros((1, 3), jnp.float32), translation[1:]], axis=0)
    x = _unit(rot6[:, 0:3])
    z = _unit(jnp.cross(x, rot6[:, 3:6]))
    y = jnp.cross(z, x)
    rot = jnp.stack([x, y, z], axis=-1)
    rmc = jnp.transpose(rot, (2, 0, 1)).reshape(3 * S, 3)
    trc = tr.T.reshape(3 * S, 1)
    rft = rot.reshape(S, 9).T
    trt = tr.T
    rftr = jnp.concatenate([rft, trt], axis=0)
    return rmc, trc, rftr


def _body(xt_ref, et_ref, w1et_ref, w1xt_ref, b1t_ref, w2t_ref, b2t_ref,
          w3t_ref, b3t_ref, rmc_ref, trc_ref, rftr_ref,
          hard_ref, soft_ref, pred_ref, cand_ref):
    xt = xt_ref[...]
    et = et_ref[...]
    h = jnp.dot(w1et_ref[...], et, preferred_element_type=jnp.float32)
    h = h + jnp.dot(w1xt_ref[...], xt, preferred_element_type=jnp.float32)
    h = jax.nn.relu(h + b1t_ref[...])
    h = jax.nn.relu(jnp.dot(w2t_ref[...], h, preferred_element_type=jnp.float32)
                    + b2t_ref[...])
    lt = jnp.dot(w3t_ref[...], h, preferred_element_type=jnp.float32) + b3t_ref[...]

    m = jnp.max(lt, axis=0, keepdims=True)
    ex = jnp.exp(lt - m)
    soft_t = ex * (1.0 / jnp.sum(ex, axis=0, keepdims=True))
    srow = jax.lax.broadcasted_iota(jnp.int32, (S, B), 0)
    ismax = lt >= m
    idx = jnp.min(jnp.where(ismax, srow, S), axis=0, keepdims=True)
    hard_t = (srow == idx).astype(jnp.float32)
    hard_ref[...] = hard_t
    soft_ref[...] = soft_t

    cand_ref[...] = (jnp.dot(rmc_ref[...], xt, preferred_element_type=jnp.float32)
                     + trc_ref[...])

    rotsel = jnp.dot(rftr_ref[...], hard_t, preferred_element_type=jnp.float32)
    pred = rotsel[9:12, :]
    for d in range(3):
        pred = pred + jnp.broadcast_to(xt[d:d + 1, :], (3, B)) * rotsel[3 * d:3 * d + 3, :]
    pred_ref[...] = pred


def kernel(xyz_cnc, xyz_cnc_embedded, W1, b1, W2, b2, W3, b3, rotation, translation):
    rmc, trc, rftr = _slot_mats(rotation, translation)
    grid = (pl.cdiv(N, B),)

    def data_spec(rows):
        return pl.BlockSpec((rows, B), lambda i: (0, i))

    def full_spec(shape):
        return pl.BlockSpec(shape, lambda i: (0,) * len(shape))

    out = pl.pallas_call(
        _body,
        grid=grid,
        in_specs=[
            data_spec(3),
            data_spec(FEAT),
            full_spec((HID, FEAT)),
            full_spec((HID, 3)),
            full_spec((HID, 1)),
            full_spec((HID, HID)),
            full_spec((HID, 1)),
            full_spec((S, HID)),
            full_spec((S, 1)),
            full_spec((3 * S, 3)),
            full_spec((3 * S, 1)),
            full_spec((12, S)),
        ],
        out_specs=[
            data_spec(S),
            data_spec(S),
            data_spec(3),
            data_spec(3 * S),
        ],
        out_shape=[
            jax.ShapeDtypeStruct((S, N), jnp.float32),
            jax.ShapeDtypeStruct((S, N), jnp.float32),
            jax.ShapeDtypeStruct((3, N), jnp.float32),
            jax.ShapeDtypeStruct((3 * S, N), jnp.float32),
        ],
        compiler_params=pltpu.CompilerParams(
            dimension_semantics=("parallel",),
        ),
    )(xyz_cnc.T, xyz_cnc_embedded.T, W1[:FEAT].T, W1[FEAT:].T,
      b1.reshape(HID, 1), W2.T, b2.reshape(HID, 1), W3.T, b3.reshape(S, 1),
      rmc, trc, rftr)

    hard_t, soft_t, pred_t, cand_c = out
    attn_hard = hard_t.T
    attn_soft = soft_t.T
    pred = pred_t.T
    cand = jnp.transpose(cand_c.reshape(3, S, N), (2, 1, 0))
    return (attn_hard, attn_soft, pred, cand)
